# baseline (speedup 1.0000x reference)
import functools

import jax
import jax.numpy as jnp
from jax import lax
from jax.experimental import pallas as pl
from jax.experimental.pallas import tpu as pltpu
from jax.experimental.pallas import tpu_sc as plsc

SCALE = 8.0
_L = 16


def _iota16():
    return lax.iota(jnp.int32, _L)


@functools.lru_cache(maxsize=None)
def _build_transpose_tc(V: int, F: int):
    CB = 512
    NBLK = -(-V // CB)
    VP2 = NBLK * (CB // 2)

    def body(in_ref, out_ref):
        t = jnp.transpose(in_ref[...])
        out_ref[...] = jnp.concatenate(
            [t[: CB // 2], t[CB // 2:]], axis=1)

    return pl.pallas_call(
        body,
        grid=(NBLK,),
        in_specs=[pl.BlockSpec((F, CB), lambda i: (0, i))],
        out_specs=pl.BlockSpec((CB // 2, 2 * F), lambda i: (i, 0)),
        out_shape=jax.ShapeDtypeStruct((VP2, 2 * F), jnp.float32),
    )


@functools.lru_cache(maxsize=None)
def _build_transpose(V: int, F: int):
    info = plsc.get_sparse_core_info()
    NW = info.num_cores * info.num_subcores
    CB = 128
    NBF = V // CB
    NB = NBF + 1
    n_iter = -(-NB // NW)
    n_pairs = n_iter // 2
    mesh = plsc.VectorSubcoreMesh(core_axis_name="c", subcore_axis_name="s")

    @functools.partial(
        pl.kernel,
        out_type=jax.ShapeDtypeStruct((V // 2, 2 * F), jnp.float32),
        mesh=mesh,
        compiler_params=pltpu.CompilerParams(needs_layout_passes=False),
        scratch_types=[
            pltpu.VMEM((F, CB), jnp.float32),
            pltpu.VMEM((F, CB), jnp.float32),
            pltpu.VMEM((CB // 2, 2 * F + 1), jnp.float32),
            pltpu.SemaphoreType.DMA,
            pltpu.SemaphoreType.DMA,
        ],
    )
    def tpose(tt_hbm, tail_hbm, out_hbm, in0, in1, obuf, sem0, sem1):
        wid = lax.axis_index("s") * info.num_cores + lax.axis_index("c")
        ins = (in0, in1)
        sems = (sem0, sem1)
        iota = _iota16()
        row_half = lax.shift_right_logical(iota, 1)
        col_half = lax.mul(lax.rem(iota, 2), F)

        def blk(k):
            return wid + NW * k

        def start_in(k, b):
            bk = blk(k)

            @pl.when(bk < NBF)
            def _():
                pltpu.async_copy(
                    tt_hbm.at[:, pl.ds(bk * CB, CB)], ins[b], sems[b])

            @pl.when(bk == NBF)
            def _():
                pltpu.async_copy(tail_hbm, ins[b], sems[b])

        def transpose_block(src, ncols):
            for g in range(ncols // _L):
                rows = row_half + (8 * g)

                @plsc.parallel_loop(0, F, 1, unroll=16)
                def _(d):
                    cols = col_half + d
                    vals = src[d, pl.ds(g * _L, _L)]
                    plsc.store_scatter(obuf, [rows, cols], vals)

        def process(k, b):
            bk = blk(k)

            @pl.when(bk < NBF)
            def _():
                pltpu.make_async_copy(
                    tt_hbm.at[:, pl.ds(bk * CB, CB)], ins[b], sems[b]).wait()
                transpose_block(ins[b], CB)
                pltpu.sync_copy(
                    obuf.at[:, pl.ds(0, 2 * F)],
                    out_hbm.at[pl.ds(bk * (CB // 2), CB // 2), :])

            @pl.when(bk == NBF)
            def _():
                pltpu.make_async_copy(tail_hbm, ins[b], sems[b]).wait()
                pltpu.sync_copy(
                    ins[b], out_hbm.at[pl.ds(V // 2 - F, F), :])

        start_in(0, 0)
        start_in(1, 1)

        def pair(g2, _):
            for b in range(2):
                k = 2 * g2 + b
                process(k, b)
                start_in(k + 2, b)
            return _

        lax.fori_loop(0, n_pairs, pair, 0)
        if n_iter % 2:
            process(2 * n_pairs, 0)

    return tpose


@functools.lru_cache(maxsize=None)
def _build_gather(S: int, B: int, V: int, F: int):
    info = plsc.get_sparse_core_info()
    NW = info.num_cores * info.num_subcores
    CH = B // NW
    n_pairs = S // 2
    mesh = plsc.VectorSubcoreMesh(core_axis_name="c", subcore_axis_name="s")

    @functools.partial(
        pl.kernel,
        out_type=jax.ShapeDtypeStruct((S, F, B), jnp.float32),
        mesh=mesh,
        compiler_params=pltpu.CompilerParams(needs_layout_passes=False),
        scratch_types=[
            pltpu.VMEM((S, CH), jnp.int32),
            pltpu.VMEM((S, CH), jnp.int32),
            pltpu.VMEM((CH, 2 * F + 1), jnp.float32),
            pltpu.VMEM((CH, 2 * F + 1), jnp.float32),
            pltpu.VMEM((F, CH), jnp.float32),
            pltpu.SemaphoreType.DMA,
            pltpu.SemaphoreType.DMA,
        ],
    )
    def emb(xt_hbm, t_hbm, out_hbm, idx_v, idx2_v, g0, g1, obuf, sem0, sem1):
        wid = lax.axis_index("s") * info.num_cores + lax.axis_index("c")
        i0 = wid * CH
        gbufs = (g0, g1)
        sems = (sem0, sem1)
        iota = _iota16()

        pltpu.sync_copy(xt_hbm.at[:, pl.ds(i0, CH)], idx_v)

        @plsc.parallel_loop(0, S, 1, unroll=2)
        def _(j):
            for g in range(CH // _L):
                sl = pl.ds(g * _L, _L)
                v = idx_v[j, sl]
                hi = lax.bitwise_and(lax.shift_right_logical(v, 1), -256)
                idx2_v[j, sl] = lax.bitwise_or(hi, lax.bitwise_and(v, 255))

        def start_gather(j, b):
            pltpu.async_copy(
                t_hbm.at[idx2_v.at[j]],
                gbufs[b].at[:, pl.ds(0, 2 * F)], sems[b])

        def process(j, b):
            pltpu.make_async_copy(
                t_hbm.at[idx2_v.at[j]],
                gbufs[b].at[:, pl.ds(0, 2 * F)], sems[b]).wait()
            src = gbufs[b]
            for g in range(CH // _L):
                rows = iota + (g * _L)
                half = lax.mul(lax.shift_right_logical(
                    idx_v[j, pl.ds(g * _L, _L)], 8) & 1, F)

                @plsc.parallel_loop(0, F, 1, unroll=16)
                def _(d):
                    cols = half + d
                    vals = plsc.load_gather(src, [rows, cols])
                    obuf[d, pl.ds(g * _L, _L)] = vals * SCALE
            pltpu.sync_copy(obuf, out_hbm.at[j, :, pl.ds(i0, CH)])

        start_gather(0, 0)
        start_gather(1, 1)

        def pair(g2, _):
            for b in range(2):
                j = 2 * g2 + b
                process(j, b)
                start_gather(j + 2, b)
            return _

        lax.fori_loop(0, n_pairs - 1, pair, 0)
        process(S - 2, 0)
        process(S - 1, 1)

    return emb


def kernel(x, table):
    B0, B1 = x.shape
    V, F = table.shape
    xt = x.T.astype(jnp.int32)
    tt = table.T
    t128 = _build_transpose_tc(V, F)(tt)
    out3 = _build_gather(B1, B0, V, F)(xt, t128)
    return jnp.transpose(out3, (2, 0, 1))


# speedup vs baseline: 1.5289x; 1.5289x over previous
import functools

import jax
import jax.numpy as jnp
from jax import lax
from jax.experimental import pallas as pl
from jax.experimental.pallas import tpu as pltpu
from jax.experimental.pallas import tpu_sc as plsc

D_MODEL = 64
SCALE = 8.0

_LANES = 16
_CH = 128


@functools.lru_cache(maxsize=None)
def _build(n_ch_total: int, V: int, D: int):
    info = plsc.get_sparse_core_info()
    NC, NS = info.num_cores, info.num_subcores
    NW = NC * NS
    n_ch_w = n_ch_total // NW
    n_pairs = n_ch_w // 2
    B = n_ch_total * _CH
    mesh = plsc.VectorSubcoreMesh(core_axis_name="c", subcore_axis_name="s")

    @functools.partial(
        pl.kernel,
        out_type=jax.ShapeDtypeStruct((B, D), jnp.float32),
        mesh=mesh,
        compiler_params=pltpu.CompilerParams(use_tc_tiling_on_sc=False),
        scratch_types=[
            pltpu.VMEM((n_ch_w, _CH), jnp.int32),
            pltpu.VMEM((_CH, D), jnp.float32),
            pltpu.VMEM((_CH, D), jnp.float32),
            pltpu.SemaphoreType.DMA,
            pltpu.SemaphoreType.DMA,
        ],
    )
    def emb(idx_hbm, table_hbm, out_hbm, idx_v, buf0, buf1, sem0, sem1):
        wid = lax.axis_index("s") * NC + lax.axis_index("c")
        row0 = wid * n_ch_w
        pltpu.sync_copy(idx_hbm.at[pl.ds(row0, n_ch_w)], idx_v)

        bufs = (buf0, buf1)
        sems = (sem0, sem1)

        def start_gather(c, b):
            pltpu.async_copy(table_hbm.at[idx_v.at[c]], bufs[b], sems[b])

        def finish_chunk(c, b):
            pltpu.make_async_copy(
                table_hbm.at[idx_v.at[c]], bufs[b], sems[b]).wait()
            buf = bufs[b]

            @plsc.parallel_loop(0, _CH, 1, unroll=4)
            def _(r):
                for j in range(D // _LANES):
                    sl = pl.ds(j * _LANES, _LANES)
                    buf[r, sl] = buf[r, sl] * SCALE

            pltpu.sync_copy(buf, out_hbm.at[pl.ds((row0 + c) * _CH, _CH)])

        start_gather(0, 0)
        start_gather(1, 1)

        def pair(g, _):
            for b in range(2):
                c = 2 * g + b
                finish_chunk(c, b)
                start_gather(c + 2, b)
            return _

        lax.fori_loop(0, n_pairs - 1, pair, 0)
        finish_chunk(n_ch_w - 2, 0)
        finish_chunk(n_ch_w - 1, 1)

    return emb


def kernel(x, table):
    B0, B1 = x.shape
    V, D = table.shape
    B = B0 * B1
    n_ch_total = B // _CH
    xf = x.reshape(n_ch_total, _CH).astype(jnp.int32)
    out = _build(n_ch_total, V, D)(xf, table)
    return out.reshape(B0, B1, D)


# speedup vs baseline: 1.5846x; 1.0364x over previous
import functools

import jax
import jax.numpy as jnp
from jax import lax
from jax.experimental import pallas as pl
from jax.experimental.pallas import tpu as pltpu
from jax.experimental.pallas import tpu_sc as plsc

D_MODEL = 64
SCALE = 8.0

_LANES = 16
_CH = 128


@functools.lru_cache(maxsize=None)
def _build(n_ch_total: int, V: int, D: int):
    info = plsc.get_sparse_core_info()
    NC, NS = info.num_cores, info.num_subcores
    NW = NC * NS
    n_ch_w = n_ch_total // NW
    n_pairs = n_ch_w // 2
    B = n_ch_total * _CH
    mesh = plsc.VectorSubcoreMesh(core_axis_name="c", subcore_axis_name="s")

    @functools.partial(
        pl.kernel,
        out_type=jax.ShapeDtypeStruct((B, D), jnp.float32),
        mesh=mesh,
        compiler_params=pltpu.CompilerParams(use_tc_tiling_on_sc=False),
        scratch_types=[
            pltpu.VMEM((n_ch_w, _CH), jnp.int32),
            pltpu.VMEM((_CH, D), jnp.float32),
            pltpu.VMEM((_CH, D), jnp.float32),
            pltpu.VMEM((_CH, D), jnp.float32),
            pltpu.VMEM((_CH, D), jnp.float32),
            pltpu.SemaphoreType.DMA,
            pltpu.SemaphoreType.DMA,
            pltpu.SemaphoreType.DMA,
            pltpu.SemaphoreType.DMA,
            pltpu.SemaphoreType.DMA,
            pltpu.SemaphoreType.DMA,
            pltpu.SemaphoreType.DMA,
            pltpu.SemaphoreType.DMA,
        ],
    )
    def emb(idx_hbm, table_hbm, out_hbm, idx_v,
            buf0, buf1, buf2, buf3,
            g0, g1, g2, g3, o0, o1, o2, o3):
        wid = lax.axis_index("s") * NC + lax.axis_index("c")
        row0 = wid * n_ch_w
        pltpu.sync_copy(idx_hbm.at[pl.ds(row0, n_ch_w)], idx_v)

        bufs = (buf0, buf1, buf2, buf3)
        gsems = (g0, g1, g2, g3)
        osems = (o0, o1, o2, o3)

        def start_gather(c, b):
            pltpu.async_copy(table_hbm.at[idx_v.at[c]], bufs[b], gsems[b])

        def out_slice(c):
            return out_hbm.at[pl.ds((row0 + c) * _CH, _CH)]

        def scale_and_store(c, b):
            pltpu.make_async_copy(
                table_hbm.at[idx_v.at[c]], bufs[b], gsems[b]).wait()
            buf = bufs[b]

            @plsc.parallel_loop(0, _CH, 1, unroll=4)
            def _(r):
                for j in range(D // _LANES):
                    sl = pl.ds(j * _LANES, _LANES)
                    buf[r, sl] = buf[r, sl] * SCALE

            pltpu.async_copy(buf, out_slice(c), osems[b])

        def wait_out(c, b):
            pltpu.make_async_copy(bufs[b], out_slice(c), osems[b]).wait()

        start_gather(0, 0)
        start_gather(1, 1)
        start_gather(2, 2)
        scale_and_store(0, 0)
        start_gather(3, 3)

        def quad(q, _):
            for t in range(4):
                c = 1 + 4 * q + t
                b = (1 + t) % 4
                scale_and_store(c, b)
                bn = t % 4
                wait_out(c - 1, bn)
                start_gather(c + 3, bn)
            return _

        lax.fori_loop(0, (n_ch_w - 4) // 4, quad, 0)
        for c in range(n_ch_w - 3, n_ch_w):
            scale_and_store(c, c % 4)
        for c in range(n_ch_w - 4, n_ch_w):
            wait_out(c, c % 4)

    return emb


def kernel(x, table):
    B0, B1 = x.shape
    V, D = table.shape
    B = B0 * B1
    n_ch_total = B // _CH
    xf = x.reshape(n_ch_total, _CH).astype(jnp.int32)
    out = _build(n_ch_total, V, D)(xf, table)
    return out.reshape(B0, B1, D)
